# Initial kernel scaffold; baseline (speedup 1.0000x reference)
import jax
import jax.numpy as jnp
from jax.experimental import pallas as pl


def kernel(x, edge_index, batch, W1, a1_src, a1_dst, b1, W2, a2_src, a2_dst, b2, W_fc1, b_fc1, W_fc2, b_fc2):
    raise NotImplementedError("write your pallas kernel here")


# speedup vs baseline: 25.1293x; 25.1293x over previous
import jax
import jax.numpy as jnp
from jax import lax
from jax.experimental import pallas as pl
from jax.experimental.pallas import tpu as pltpu
from jax.experimental.pallas import tpu_sc as plsc

N = 10000
E = 320000
D = 128
H = 128
NG = 64
NC = 10

NCORES = 2
NSUB = 16
NW = NCORES * NSUB
EPW = E // NW
CHUNK = 80
NCHUNK = EPW // CHUNK
RPT = 640
NPAD = RPT * NSUB
ZR = 128

SELU_SCALE = 1.0507009873554805
SELU_ALPHA = 1.6732632423543772


def _selu(v):
    return SELU_SCALE * jnp.where(v > 0, v, SELU_ALPHA * (jnp.exp(v) - 1.0))


def _proj_body(x_ref, w_ref, a2_ref, h_ref, asd_ref):
    h = jnp.dot(x_ref[...], w_ref[...], preferred_element_type=jnp.float32)
    h_ref[...] = h
    asd_ref[...] = jnp.dot(h, a2_ref[...], preferred_element_type=jnp.float32)


def _project(x, W, A2):
    R = 1000
    return pl.pallas_call(
        _proj_body,
        grid=(N // R,),
        in_specs=[
            pl.BlockSpec((R, D), lambda i: (i, 0)),
            pl.BlockSpec((D, H), lambda i: (0, 0)),
            pl.BlockSpec((H, 128), lambda i: (0, 0)),
        ],
        out_specs=[
            pl.BlockSpec((R, H), lambda i: (i, 0)),
            pl.BlockSpec((R, 128), lambda i: (i, 0)),
        ],
        out_shape=[
            jax.ShapeDtypeStruct((N, H), jnp.float32),
            jax.ShapeDtypeStruct((N, 128), jnp.float32),
        ],
    )(x, W, A2)


def _edge_body(h_hbm, as_hbm, ad_hbm, src_hbm, dst_hbm, acc_hbm, den_hbm,
               acc_s, den_s, as_v, ad_v, src_v, dst_v, rows_v, ee_v,
               zrow_v, zden_v, sem):
    c = lax.axis_index("c")
    s = lax.axis_index("s")
    w = s * NCORES + c

    pltpu.sync_copy(as_hbm, as_v)
    pltpu.sync_copy(ad_hbm, ad_v)

    zero16 = jnp.zeros((16,), jnp.float32)

    def zrow_it(r, carry):
        for k in range(8):
            zrow_v[r, pl.ds(k * 16, 16)] = zero16
        return carry

    lax.fori_loop(0, ZR, zrow_it, 0)

    def zden_it(i, carry):
        zden_v[pl.ds(i * 16, 16)] = zero16
        return carry

    lax.fori_loop(0, RPT // 16, zden_it, 0)

    for j in range(5):
        pltpu.sync_copy(zrow_v, acc_s.at[pl.ds(s * RPT + j * ZR, ZR)])
    pltpu.sync_copy(zden_v, den_s.at[pl.ds(s * RPT, RPT)])
    plsc.subcore_barrier()

    def chunk_it(i, carry):
        base = pl.multiple_of(w * EPW + i * CHUNK, CHUNK)
        pltpu.sync_copy(src_hbm.at[pl.ds(base, CHUNK)], src_v)
        pltpu.sync_copy(dst_hbm.at[pl.ds(base, CHUNK)], dst_v)
        pltpu.async_copy(h_hbm.at[src_v], rows_v, sem).wait()
        for j in range(CHUNK // 16):
            si = src_v[pl.ds(j * 16, 16)]
            di = dst_v[pl.ds(j * 16, 16)]
            t = plsc.load_gather(as_v, [si]) + plsc.load_gather(ad_v, [di])
            t = jnp.where(t >= 0, t, t * jnp.float32(0.2))
            ee_v[pl.ds(j * 16, 16)] = jnp.exp(t)

        def scale_it(j2, carry2):
            ee16 = ee_v[pl.ds(j2 * 16, 16)]
            for l in range(16):
                sc = ee16[l]
                e = j2 * 16 + l
                for k in range(8):
                    rows_v[e, pl.ds(k * 16, 16)] = (
                        rows_v[e, pl.ds(k * 16, 16)] * sc)
            return carry2

        lax.fori_loop(0, CHUNK // 16, scale_it, 0)
        pltpu.sync_copy(rows_v, acc_s.at[dst_v], add=True)
        pltpu.sync_copy(ee_v, den_s.at[dst_v], add=True)
        return carry

    lax.fori_loop(0, NCHUNK, chunk_it, 0)
    plsc.subcore_barrier()

    r0 = s * RPT
    pltpu.sync_copy(acc_s.at[pl.ds(r0, RPT)], acc_hbm.at[c, pl.ds(r0, RPT)])
    pltpu.sync_copy(den_s.at[pl.ds(s * RPT, RPT)], den_hbm.at[c, s])


def _edge_pass(h, as_, ad, src, dst):
    mesh = plsc.VectorSubcoreMesh(
        core_axis_name="c", subcore_axis_name="s",
        num_cores=NCORES, num_subcores=NSUB)
    fn = pl.kernel(
        _edge_body,
        out_type=[
            jax.ShapeDtypeStruct((NCORES, NPAD, H), jnp.float32),
            jax.ShapeDtypeStruct((NCORES, NSUB, RPT), jnp.float32),
        ],
        mesh=mesh,
        compiler_params=pltpu.CompilerParams(needs_layout_passes=False),
        scratch_types=[
            pltpu.VMEM_SHARED((NPAD, H), jnp.float32),
            pltpu.VMEM_SHARED((NPAD,), jnp.float32),
            pltpu.VMEM((N,), jnp.float32),
            pltpu.VMEM((N,), jnp.float32),
            pltpu.VMEM((CHUNK,), jnp.int32),
            pltpu.VMEM((CHUNK,), jnp.int32),
            pltpu.VMEM((CHUNK, H), jnp.float32),
            pltpu.VMEM((CHUNK,), jnp.float32),
            pltpu.VMEM((ZR, H), jnp.float32),
            pltpu.VMEM((RPT,), jnp.float32),
            pltpu.SemaphoreType.DMA,
        ],
    )
    return fn(h, as_, ad, src, dst)


def _combine_body(h_ref, asd_ref, acc_ref, den_ref, b_ref, o_ref):
    h = h_ref[...]
    asd = asd_ref[...]
    t = asd[:, 0] + asd[:, 1]
    t = jnp.where(t >= 0, t, t * jnp.float32(0.2))
    es = jnp.exp(t)
    num = acc_ref[0] + acc_ref[1] + es[:, None] * h
    den = den_ref[0, 0, 0] + den_ref[1, 0, 0] + es + jnp.float32(1e-16)
    o_ref[...] = _selu(num / den[:, None] + b_ref[0])


def _combine(h, asd, acc, den4, b):
    R = 1000
    return pl.pallas_call(
        _combine_body,
        grid=(N // R,),
        in_specs=[
            pl.BlockSpec((R, H), lambda i: (i, 0)),
            pl.BlockSpec((R, 128), lambda i: (i, 0)),
            pl.BlockSpec((NCORES, R, H), lambda i: (0, i, 0)),
            pl.BlockSpec((NCORES, 1, 1, R), lambda i: (0, i, 0, 0)),
            pl.BlockSpec((1, H), lambda i: (0, 0)),
        ],
        out_specs=pl.BlockSpec((R, H), lambda i: (i, 0)),
        out_shape=jax.ShapeDtypeStruct((N, H), jnp.float32),
    )(h, asd, acc, den4, b)


def _pool_body(hb_ref, bt_ref, sum_ref, cnt_ref):
    i = pl.program_id(0)

    @pl.when(i == 0)
    def _():
        sum_ref[...] = jnp.zeros_like(sum_ref)
        cnt_ref[...] = jnp.zeros_like(cnt_ref)

    b = bt_ref[0, 0]
    gids = lax.broadcasted_iota(jnp.int32, (1, NG), 1)
    oh = (b[:, None] == gids).astype(jnp.float32)
    sum_ref[...] += lax.dot_general(
        oh, hb_ref[...], (((0,), (0,)), ((), ())),
        preferred_element_type=jnp.float32)
    cnt_ref[...] += jnp.sum(oh, axis=0)[:, None]


def _pool(h, batch3):
    RB = 400
    return pl.pallas_call(
        _pool_body,
        grid=(N // RB,),
        in_specs=[
            pl.BlockSpec((RB, H), lambda i: (i, 0)),
            pl.BlockSpec((1, 1, RB), lambda i: (i, 0, 0)),
        ],
        out_specs=[
            pl.BlockSpec((NG, H), lambda i: (0, 0)),
            pl.BlockSpec((NG, H), lambda i: (0, 0)),
        ],
        out_shape=[
            jax.ShapeDtypeStruct((NG, H), jnp.float32),
            jax.ShapeDtypeStruct((NG, H), jnp.float32),
        ],
    )(h, batch3)


def _mlp_body(sum_ref, cnt_ref, w1_ref, b1_ref, w2_ref, b2_ref, o_ref):
    g = sum_ref[...] / jnp.maximum(cnt_ref[...], 1.0)
    g = _selu(g)
    g1 = _selu(jnp.dot(g, w1_ref[...], preferred_element_type=jnp.float32)
               + b1_ref[0])
    lg = (jnp.dot(g1, w2_ref[...], preferred_element_type=jnp.float32)
          + b2_ref[0])
    lane = lax.broadcasted_iota(jnp.int32, (NG, 128), 1)
    lg = jnp.where(lane < NC, lg, jnp.float32(-1e30))
    m = jnp.max(lg, axis=1, keepdims=True)
    lse = jnp.log(jnp.sum(jnp.exp(lg - m), axis=1, keepdims=True)) + m
    o_ref[...] = lg - lse


def _mlp(sums, cnt, W1p, b1p, W2p, b2p):
    return pl.pallas_call(
        _mlp_body,
        in_specs=[
            pl.BlockSpec((NG, H), lambda: (0, 0)),
            pl.BlockSpec((NG, H), lambda: (0, 0)),
            pl.BlockSpec((H, 128), lambda: (0, 0)),
            pl.BlockSpec((1, 128), lambda: (0, 0)),
            pl.BlockSpec((128, 128), lambda: (0, 0)),
            pl.BlockSpec((1, 128), lambda: (0, 0)),
        ],
        out_specs=pl.BlockSpec((NG, 128), lambda: (0, 0)),
        out_shape=jax.ShapeDtypeStruct((NG, 128), jnp.float32),
    )(sums, cnt, W1p, b1p, W2p, b2p)


def _gat_layer(xin, W, a_src, a_dst, b, src, dst):
    A2 = jnp.concatenate(
        [a_src[:, None], a_dst[:, None], jnp.zeros((H, 126), jnp.float32)],
        axis=1)
    h, asd = _project(xin, W, A2)
    as_ = asd[:, 0] + 0.0
    ad = asd[:, 1] + 0.0
    acc, den = _edge_pass(h, as_, ad, src, dst)
    den4 = den.reshape(NCORES, NPAD)[:, :N].reshape(NCORES, N // 1000, 1, 1000)
    return _combine(h, asd, acc, den4, b[None, :])


def kernel(x, edge_index, batch, W1, a1_src, a1_dst, b1, W2, a2_src, a2_dst,
           b2, W_fc1, b_fc1, W_fc2, b_fc2):
    src = edge_index[0] + 0
    dst = edge_index[1] + 0

    x2 = _gat_layer(x, W1, a1_src, a1_dst, b1, src, dst)
    x3 = _gat_layer(x2, W2, a2_src, a2_dst, b2, src, dst)

    batch3 = batch.reshape(N // 400, 1, 400)
    sums, cnt = _pool(x3, batch3)

    W1p = jnp.concatenate([W_fc1, jnp.zeros((H, 128 - NG), jnp.float32)], axis=1)
    b1p = jnp.concatenate([b_fc1, jnp.zeros((128 - NG,), jnp.float32)])[None, :]
    W2p = jnp.zeros((128, 128), jnp.float32).at[:NG, :NC].set(W_fc2)
    b2p = jnp.concatenate([b_fc2, jnp.zeros((128 - NC,), jnp.float32)])[None, :]
    out = _mlp(sums, cnt, W1p, b1p, W2p, b2p)
    return out[:, :NC]
